# baseline (speedup 1.0000x reference)
import functools

import jax
import jax.numpy as jnp
from jax import lax
from jax.experimental import pallas as pl
from jax.experimental.pallas import tpu as pltpu
from jax.experimental.pallas import tpu_sc as plsc

H = 32
W = 32
D = 128
BS = 16
NC = 2
NS = 16
L = 16
CPT = (2 * D) // NS


def _pos_body(row_hbm, col_hbm, out_hbm, tab, chunk, sem):
    cid = lax.axis_index("c")
    sid = lax.axis_index("s")

    pltpu.sync_copy(col_hbm.at[pl.ds(0, H * D)], tab.at[pl.ds(0, H * D)])
    pltpu.sync_copy(row_hbm.at[pl.ds(0, H * D)], tab.at[pl.ds(H * D, H * D)])

    iota = lax.iota(jnp.int32, L)

    for j in range(CPT):
        ch = sid * CPT + j
        is_col = ch < D
        iscolv = jnp.full((L,), is_col)
        bvec = jnp.full((L,), lax.rem(ch, D), dtype=jnp.int32)

        def ybody(y, carry):
            arow = jnp.full((L,), H + y, dtype=jnp.int32)
            a0 = jnp.where(iscolv, iota, arow)
            a1 = jnp.where(iscolv, iota + L, arow)
            v0 = plsc.load_gather(tab, [a0 * D + bvec])
            v1 = plsc.load_gather(tab, [a1 * D + bvec])
            chunk[j, pl.ds(y * W, L)] = v0
            chunk[j, pl.ds(y * W + L, L)] = v1
            return carry

        lax.fori_loop(0, H, ybody, 0)

    copies = []
    for b in range(BS // NC):
        bb = cid * (BS // NC) + b
        copies.append(
            pltpu.async_copy(chunk, out_hbm.at[bb, pl.ds(sid * CPT, CPT)], sem)
        )
    for c in copies:
        c.wait()


@jax.jit
def _pos_embed(row_embed, col_embed):
    mesh = plsc.VectorSubcoreMesh(
        core_axis_name="c", subcore_axis_name="s", num_cores=NC, num_subcores=NS
    )
    return pl.kernel(
        _pos_body,
        out_type=jax.ShapeDtypeStruct((BS, 2 * D, H * W), jnp.float32),
        mesh=mesh,
        scratch_types=[
            pltpu.VMEM((2 * H * D,), jnp.float32),
            pltpu.VMEM((CPT, H * W), jnp.float32),
            pltpu.SemaphoreType.DMA,
        ],
        compiler_params=pltpu.CompilerParams(needs_layout_passes=False),
    )(row_embed.reshape(-1), col_embed.reshape(-1))


def kernel(mask, row_embed, col_embed):
    bs, h, w = mask.shape
    pos = _pos_embed(row_embed, col_embed)
    return pos.reshape(bs, 2 * D, h, w)
